# baseline (speedup 1.0000x reference)
import jax
import jax.numpy as jnp
from jax.experimental import pallas as pl
from jax.experimental.pallas import tpu as pltpu


def _copy_body(idx_ref, kv_ref, out_ref):
    out_ref[0, 0, 0] = kv_ref[0, 0]


def kernel(r_idx, kv):
    b, p2, w2, c_kv = kv.shape
    topk = r_idx.shape[2]

    grid_spec = pltpu.PrefetchScalarGridSpec(
        num_scalar_prefetch=1,
        grid=(b, p2, topk),
        in_specs=[
            pl.BlockSpec(
                (1, 1, w2, c_kv),
                lambda bi, qi, ki, idx_ref: (bi, idx_ref[bi, qi, ki], 0, 0),
            )
        ],
        out_specs=pl.BlockSpec(
            (1, 1, 1, w2, c_kv),
            lambda bi, qi, ki, idx_ref: (bi, qi, ki, 0, 0),
        ),
    )

    return pl.pallas_call(
        _copy_body,
        grid_spec=grid_spec,
        out_shape=jax.ShapeDtypeStruct((b, p2, topk, w2, c_kv), kv.dtype),
    )(r_idx.astype(jnp.int32), kv)
